# baseline (speedup 1.0000x reference)
import jax
import jax.numpy as jnp
import numpy as np
from jax import lax
from jax.experimental import pallas as pl
from jax.experimental.pallas import tpu as pltpu
from jax.experimental.pallas import tpu_sc as plsc

_L = 8
_T = 524288
_MASK = _T - 1
_P1 = 2654435761 - (1 << 32)
_P2 = 805459861
_NW = 32
_P = 64
_G = _P // 16

_RES = tuple(int(16 * 1.5 ** l) for l in range(_L))
_M2 = (17, 25, 37)
_LOFF = (0, 296, 928)


def _enc_body(zin, yin, xin, tvin, t0, t1, t2, t3, t4, enc,
              cz, cy, cx,
              sg1, sg2, sg3, sgi0, sgi1, sgi2,
              idx04a, r0a, r4a, idx1a, idx2a, idx3a, r1a, r2a, r3a,
              idx04b, r0b, r4b, idx1b, idx2b, idx3b, r1b, r2b, r3b,
              encb, tvv,
              s0a, s4a, s1a, s2a, s3a, s0b, s4b, s1b, s2b, s3b):
    n = zin.shape[0]
    ptsw = n // _NW
    chunks = ptsw // _P
    cid = lax.axis_index("c")
    sid = lax.axis_index("s")
    wid = sid * 2 + cid
    pltpu.sync_copy(tvin, tvv)
    iota = lax.iota(jnp.int32, 16)

    for l, sgi in ((0, sgi0), (1, sgi1), (2, sgi2)):
        m = _M2[l]
        lb2 = 2 * l * _T

        def gen2(a, carry, m=m, sgi=sgi, lb2=lb2):
            av = lax.broadcast(a, (16,))
            for j in range(-(-m // 16)):
                b = j * 16 + iota
                e0 = (((av ^ (b * _P1)) & _MASK) << 1) + lb2
                pos = (a * m + b) << 1
                msk = b < m
                plsc.store_scatter(sgi, [pos], e0, mask=msk)
                plsc.store_scatter(sgi, [pos + 1], e0 + 1, mask=msk)
            return carry

        lax.fori_loop(0, m, gen2, 0)
    dgs = []
    for l, sgi in ((0, sgi0), (1, sgi1), (2, sgi2)):
        off2 = 2 * _LOFF[l]
        m = _M2[l]
        for tk, sg, sem in ((t1, sg1, s1a), (t2, sg2, s2a), (t3, sg3, s3a)):
            dgs.append(pltpu.async_copy(
                tk.at[sgi], sg.at[pl.ds(off2, 2 * m * m)], sem))
    for d in dgs:
        d.wait()

    def cells(g):
        zv = cz[pl.ds(g * 16, 16)]
        yv = cy[pl.ds(g * 16, 16)]
        xv = cx[pl.ds(g * 16, 16)]
        return zv, yv, xv

    bufs = (
        (idx04a, r0a, r4a, idx1a, idx2a, idx3a, r1a, r2a, r3a,
         s0a, s4a, s1a, s2a, s3a),
        (idx04b, r0b, r4b, idx1b, idx2b, idx3b, r1b, r2b, r3b,
         s0b, s4b, s1b, s2b, s3b),
    )

    def prep(l):
        (idx04, r0, r4, idx1, idx2, idx3, r1, r2, r3,
         s0, s4, s1, s2, s3) = bufs[l % 2]
        resf = jnp.float32(float(_RES[l]))
        lb2 = 2 * l * _T
        hbm2d = l >= 3

        def body(g, c2):
            zv, yv, xv = cells(g)
            zi = (zv * resf).astype(jnp.int32)
            yi = (yv * resf).astype(jnp.int32)
            xi = (xv * resf).astype(jnp.int32)
            my_ = yi * _P1
            mx_ = xi * _P2
            rows = g * 16 + iota
            for c in range(8):
                h0 = zi if (c & 1) == 0 else zi + 1
                h1 = my_ if (c & 2) == 0 else my_ + _P1
                h2 = mx_ if (c & 4) == 0 else mx_ + _P2
                e0 = (((((h0 ^ h1) ^ h2)) & _MASK) << 1) + lb2
                pos = (c * _P + rows) << 1
                plsc.store_scatter(idx04, [pos], e0)
                plsc.store_scatter(idx04, [pos + 1], e0 + 1)
            if hbm2d:
                for idref, ci, di in ((idx1, yi, xi), (idx2, xi, zi),
                                      (idx3, zi, yi)):
                    md = di * _P1
                    for c in range(4):
                        h0 = ci if (c & 1) == 0 else ci + 1
                        h1 = md if (c & 2) == 0 else md + _P1
                        e0 = (((h0 ^ h1) & _MASK) << 1) + lb2
                        pos = (c * _P + rows) << 1
                        plsc.store_scatter(idref, [pos], e0)
                        plsc.store_scatter(idref, [pos + 1], e0 + 1)
            return c2

        lax.fori_loop(0, _G, body, 0)
        ds_ = [pltpu.async_copy(t0.at[idx04], r0, s0),
               pltpu.async_copy(t4.at[idx04], r4, s4)]
        if hbm2d:
            ds_ += [pltpu.async_copy(t1.at[idx1], r1, s1),
                    pltpu.async_copy(t2.at[idx2], r2, s2),
                    pltpu.async_copy(t3.at[idx3], r3, s3)]
        return ds_

    def consume(l):
        (idx04, r0, r4, idx1, idx2, idx3, r1, r2, r3,
         s0, s4, s1, s2, s3) = bufs[l % 2]
        resf = jnp.float32(float(_RES[l]))
        staged = l < 3

        def body(g, c2):
            rows = g * 16 + iota
            erows = rows * 84
            zv, yv, xv = cells(g)
            xs = zv * resf
            zi = xs.astype(jnp.int32)
            wz = xs - zi.astype(jnp.float32)
            xs = yv * resf
            yi = xs.astype(jnp.int32)
            wy = xs - yi.astype(jnp.float32)
            xs = xv * resf
            xi = xs.astype(jnp.int32)
            wx = xs - xi.astype(jnp.float32)
            wz0 = 1.0 - wz
            wy0 = 1.0 - wy
            wx0 = 1.0 - wx
            acc = [jnp.zeros((16,), jnp.float32) for _ in range(4)]
            for c in range(8):
                wc = (wz if c & 1 else wz0) * (wy if c & 2 else wy0)
                wc = wc * (wx if c & 4 else wx0)
                pc = (c * _P + rows) << 1
                for f in range(2):
                    v = plsc.load_gather(r0, [pc + f])
                    acc[f] = acc[f] + v * wc
                for f in range(2):
                    v = plsc.load_gather(r4, [pc + f])
                    acc[2 + f] = acc[2 + f] + v * wc
            for f in range(4):
                col = l * 2 + (f & 1) + 64 * (f >> 1)
                plsc.store_scatter(encb, [erows + col], acc[f])
            if staged:
                m = _M2[l]
                loff = _LOFF[l]
                srcs = ((sg1, yi, wy, wy0, xi, wx, wx0, 16),
                        (sg2, xi, wx, wx0, zi, wz, wz0, 32),
                        (sg3, zi, wz, wz0, yi, wy, wy0, 48))
            else:
                srcs = ((r1, yi, wy, wy0, xi, wx, wx0, 16),
                        (r2, xi, wx, wx0, zi, wz, wz0, 32),
                        (r3, zi, wz, wz0, yi, wy, wy0, 48))
            for rref, ci, cw, cw0, di, dw, dw0, bcol in srcs:
                if staged:
                    b2 = (loff + ci * m + di) << 1
                a2 = [jnp.zeros((16,), jnp.float32) for _ in range(2)]
                for c in range(4):
                    wc = (cw if c & 1 else cw0) * (dw if c & 2 else dw0)
                    if staged:
                        pc = b2 + (c & 1) * 2 * m + ((c >> 1) & 1) * 2
                    else:
                        pc = (c * _P + rows) << 1
                    for f in range(2):
                        v = plsc.load_gather(rref, [pc + f])
                        a2[f] = a2[f] + v * wc
                for f in range(2):
                    col = bcol + l * 2 + f
                    plsc.store_scatter(encb, [erows + col], a2[f])
            return c2

        lax.fori_loop(0, _G, body, 0)

    def chunk_body(k, carry):
        base = wid * ptsw + k * _P
        pltpu.sync_copy(zin.at[pl.ds(base, _P)], cz)
        pltpu.sync_copy(yin.at[pl.ds(base, _P)], cy)
        pltpu.sync_copy(xin.at[pl.ds(base, _P)], cx)

        def raw_body(g, c2):
            rows = (g * 16 + iota) * 84
            zv, yv, xv = cells(g)
            tv = tvv[...]
            plsc.store_scatter(encb, [rows + 80], zv)
            plsc.store_scatter(encb, [rows + 81], yv)
            plsc.store_scatter(encb, [rows + 82], xv)
            plsc.store_scatter(encb, [rows + 83], tv)
            return c2

        lax.fori_loop(0, _G, raw_body, 0)

        pend = [None] * _L
        pend[0] = prep(0)
        pend[1] = prep(1)
        for l in range(_L):
            for d in pend[l]:
                d.wait()
            consume(l)
            if l + 2 < _L:
                pend[l + 2] = prep(l + 2)
        pltpu.sync_copy(encb, enc.at[pl.ds(base * 84, _P * 84)])
        return carry

    lax.fori_loop(0, chunks, chunk_body, 0)


def _sc_encode(zin, yin, xin, tvec, t0, t1, t2, t3, t4):
    n = zin.shape[0]
    mesh = plsc.VectorSubcoreMesh(core_axis_name="c", subcore_axis_name="s")
    n2 = 2 * (928 + 37 * 37)
    ab = []
    for _ in range(2):
        ab += [
            pltpu.VMEM((16 * _P,), jnp.int32),
            pltpu.VMEM((16 * _P,), jnp.float32),
            pltpu.VMEM((16 * _P,), jnp.float32),
            pltpu.VMEM((8 * _P,), jnp.int32),
            pltpu.VMEM((8 * _P,), jnp.int32),
            pltpu.VMEM((8 * _P,), jnp.int32),
            pltpu.VMEM((8 * _P,), jnp.float32),
            pltpu.VMEM((8 * _P,), jnp.float32),
            pltpu.VMEM((8 * _P,), jnp.float32),
        ]
    return pl.kernel(
        _enc_body,
        out_type=jax.ShapeDtypeStruct((n * 84,), jnp.float32),
        mesh=mesh,
        compiler_params=pltpu.CompilerParams(
            needs_layout_passes=False, use_tc_tiling_on_sc=False),
        scratch_types=[
            pltpu.VMEM((_P,), jnp.float32),
            pltpu.VMEM((_P,), jnp.float32),
            pltpu.VMEM((_P,), jnp.float32),
            pltpu.VMEM((n2,), jnp.float32),
            pltpu.VMEM((n2,), jnp.float32),
            pltpu.VMEM((n2,), jnp.float32),
            pltpu.VMEM((2 * _M2[0] ** 2,), jnp.int32),
            pltpu.VMEM((2 * _M2[1] ** 2,), jnp.int32),
            pltpu.VMEM((2 * _M2[2] ** 2,), jnp.int32),
            *ab,
            pltpu.VMEM((_P * 84,), jnp.float32),
            pltpu.VMEM((16,), jnp.float32),
        ] + [pltpu.SemaphoreType.DMA] * 10,
    )(zin, yin, xin, tvec, t0, t1, t2, t3, t4)


_WR = 1024


def _repack_body(t0, t1, t2, t3, t4, o0, o1, o2, o3, o4, va, vb):
    cid = lax.axis_index("c")
    sid = lax.axis_index("s")
    wid = sid * 2 + cid
    iota = lax.iota(jnp.int32, 16)
    rows_per_w = _L * _T // _NW
    lvl = wid // 4
    tstart = (wid % 4) * rows_per_w
    nch = rows_per_w // _WR

    for tin, tout in ((t0, o0), (t1, o1), (t2, o2), (t3, o3), (t4, o4)):
        def chunk(c, carry, tin=tin, tout=tout):
            ts = tstart + c * _WR
            pltpu.sync_copy(tin.at[pl.ds(lvl, 1), pl.ds(ts, _WR)], va)

            def bounce(j, c2):
                r = j * 8 + (iota >> 1)
                f = iota & 1
                v = plsc.load_gather(va, [jnp.zeros((16,), jnp.int32), r, f])
                vb[pl.ds(j * 16, 16)] = v
                return c2

            lax.fori_loop(0, 2 * _WR // 16, bounce, 0)
            pltpu.sync_copy(vb, tout.at[pl.ds((lvl * _T + ts) * 2, 2 * _WR)])
            return carry

        lax.fori_loop(0, nch, chunk, 0)


def _repack(t0, t1, t2, t3, t4):
    mesh = plsc.VectorSubcoreMesh(core_axis_name="c", subcore_axis_name="s")
    out = jax.ShapeDtypeStruct((_L * _T * 2,), jnp.float32)
    return pl.kernel(
        _repack_body,
        out_type=(out, out, out, out, out),
        mesh=mesh,
        compiler_params=pltpu.CompilerParams(
            needs_layout_passes=False, use_tc_tiling_on_sc=False),
        scratch_types=[
            pltpu.VMEM((1, _WR, 2), jnp.float32),
            pltpu.VMEM((2 * _WR,), jnp.float32),
        ],
    )(t0, t1, t2, t3, t4)


def _mlp_body(x_ref, w1_ref, w2_ref, w3_ref, o_ref):
    dn = (((1,), (0,)), ((), ()))
    h = jnp.maximum(lax.dot_general(
        x_ref[...], w1_ref[...], dn,
        precision=lax.Precision.HIGHEST,
        preferred_element_type=jnp.float32), 0.0)
    h = jnp.maximum(lax.dot_general(
        h, w2_ref[...], dn,
        precision=lax.Precision.HIGHEST,
        preferred_element_type=jnp.float32), 0.0)
    o_ref[...] = lax.dot_general(
        h, w3_ref[...], dn,
        precision=lax.Precision.HIGHEST,
        preferred_element_type=jnp.float32)


def _mlp(enc, W1, W2, W3):
    n = enc.shape[0]
    blk = 8192 if n % 8192 == 0 else n
    return pl.pallas_call(
        _mlp_body,
        out_shape=jax.ShapeDtypeStruct((n, 1), jnp.float32),
        grid=(n // blk,),
        in_specs=[
            pl.BlockSpec((blk, 84), lambda i: (i, 0)),
            pl.BlockSpec((84, 64), lambda i: (0, 0)),
            pl.BlockSpec((64, 64), lambda i: (0, 0)),
            pl.BlockSpec((64, 1), lambda i: (0, 0)),
        ],
        out_specs=pl.BlockSpec((blk, 1), lambda i: (i, 0)),
    )(enc, W1, W2, W3)


def kernel(zyx, t, table0, table1, table2, table3, table4, W1, W2, W3):
    zc = zyx.T
    zin, yin, xin = zc[0], zc[1], zc[2]
    t0f, t1f, t2f, t3f, t4f = _repack(table0, table1, table2, table3, table4)
    tvec = jnp.full((16,), t, jnp.float32)
    enc = _sc_encode(zin, yin, xin, tvec, t0f, t1f, t2f, t3f,
                     t4f).reshape(-1, 84)
    return _mlp(enc, W1, W2, W3)


# speedup vs baseline: 1.2817x; 1.2817x over previous
import jax
import jax.numpy as jnp
import numpy as np
from jax import lax
from jax.experimental import pallas as pl
from jax.experimental.pallas import tpu as pltpu
from jax.experimental.pallas import tpu_sc as plsc

_L = 8
_T = 524288
_MASK = _T - 1
_P1 = 2654435761 - (1 << 32)
_P2 = 805459861
_NW = 32
_P = 16
_G = _P // 16


def _enc_body(zin, yin, xin, tvin, t04, t1, t2, t3, enc,
              cz, cy, cx,
              idx04a, idx1a, idx2a, idx3a, idx04b, idx1b, idx2b, idx3b,
              r04a, r1a, r2a, r3a, r04b, r1b, r2b, r3b,
              encb, tvv,
              s04a, s1a, s2a, s3a, s04b, s1b, s2b, s3b):
    n = zin.shape[0]
    ptsw = n // _NW
    chunks = ptsw // _P
    cid = lax.axis_index("c")
    sid = lax.axis_index("s")
    wid = sid * 2 + cid
    pltpu.sync_copy(tvin, tvv)
    iota = lax.iota(jnp.int32, 16)

    def chunk_body(k, carry):
        base = wid * ptsw + k * _P
        pltpu.sync_copy(zin.at[pl.ds(base, _P)], cz)
        pltpu.sync_copy(yin.at[pl.ds(base, _P)], cy)
        pltpu.sync_copy(xin.at[pl.ds(base, _P)], cx)

        def raw_body(g, c2):
            rows = (g * 16 + iota) * 84
            zv = cz[pl.ds(g * 16, 16)]
            yv = cy[pl.ds(g * 16, 16)]
            xv = cx[pl.ds(g * 16, 16)]
            tv = tvv[...]
            plsc.store_scatter(encb, [rows + 80], zv)
            plsc.store_scatter(encb, [rows + 81], yv)
            plsc.store_scatter(encb, [rows + 82], xv)
            plsc.store_scatter(encb, [rows + 83], tv)
            return c2

        lax.fori_loop(0, _G, raw_body, 0)

        def idx_phase(resf, lbase, idx04, idx1, idx2, idx3):
            def idx_body(g, c2):
                zv = cz[pl.ds(g * 16, 16)]
                yv = cy[pl.ds(g * 16, 16)]
                xv = cx[pl.ds(g * 16, 16)]
                zi = (zv * resf).astype(jnp.int32)
                yi = (yv * resf).astype(jnp.int32)
                xi = (xv * resf).astype(jnp.int32)
                my_ = yi * _P1
                mx_ = xi * _P2
                for c in range(8):
                    h0 = zi if (c & 1) == 0 else zi + 1
                    h1 = my_ if (c & 2) == 0 else my_ + _P1
                    h2 = mx_ if (c & 4) == 0 else mx_ + _P2
                    idx04[pl.ds(c * _P + g * 16, 16)] = (((h0 ^ h1) ^ h2) & _MASK) + lbase
                for idref, ci, di in ((idx1, yi, xi), (idx2, xi, zi), (idx3, zi, yi)):
                    md = di * _P1
                    for c in range(4):
                        h0 = ci if (c & 1) == 0 else ci + 1
                        h1 = md if (c & 2) == 0 else md + _P1
                        idref[pl.ds(c * _P + g * 16, 16)] = ((h0 ^ h1) & _MASK) + lbase
                return c2

            lax.fori_loop(0, _G, idx_body, 0)

        def con_phase(resf, l, r04, r1, r2, r3):
            def con_body(g, c2):
                rows = g * 16 + iota
                erows = rows * 84
                zv = cz[pl.ds(g * 16, 16)]
                yv = cy[pl.ds(g * 16, 16)]
                xv = cx[pl.ds(g * 16, 16)]
                xs = zv * resf
                zi = xs.astype(jnp.int32)
                wz = xs - zi.astype(jnp.float32)
                xs = yv * resf
                yi = xs.astype(jnp.int32)
                wy = xs - yi.astype(jnp.float32)
                xs = xv * resf
                xi = xs.astype(jnp.int32)
                wx = xs - xi.astype(jnp.float32)
                wz0 = 1.0 - wz
                wy0 = 1.0 - wy
                wx0 = 1.0 - wx
                acc = [jnp.zeros((16,), jnp.float32) for _ in range(4)]
                for c in range(8):
                    wc = (wz if c & 1 else wz0) * (wy if c & 2 else wy0)
                    wc = wc * (wx if c & 4 else wx0)
                    rr = c * _P + rows
                    for f in range(4):
                        v = plsc.load_gather(r04, [rr, jnp.full((16,), f, jnp.int32)])
                        acc[f] = acc[f] + v * wc
                for f in range(4):
                    col = l * 2 + (f & 1) + 64 * (f >> 1)
                    plsc.store_scatter(encb, [erows + col], acc[f])
                for rref, cwp, dwp, bcol in (
                        (r1, (wy0, wy), (wx0, wx), 16),
                        (r2, (wx0, wx), (wz0, wz), 32),
                        (r3, (wz0, wz), (wy0, wy), 48)):
                    a2 = [jnp.zeros((16,), jnp.float32) for _ in range(2)]
                    for c in range(4):
                        wc = cwp[c & 1] * dwp[(c & 2) >> 1]
                        rr = c * _P + rows
                        for f in range(2):
                            v = plsc.load_gather(rref, [rr, jnp.full((16,), f, jnp.int32)])
                            a2[f] = a2[f] + v * wc
                    for f in range(2):
                        col = bcol + l * 2 + f
                        plsc.store_scatter(encb, [erows + col], a2[f])
                return c2

            lax.fori_loop(0, _G, con_body, 0)

        def pair_body(j, p):
            l0 = 2 * j
            l1 = 2 * j + 1
            p1 = p * 1.5
            res0 = lax.convert_element_type(
                lax.convert_element_type(p, jnp.int32), jnp.float32)
            res1 = lax.convert_element_type(
                lax.convert_element_type(p1, jnp.int32), jnp.float32)
            idx_phase(res0, l0 * _T, idx04a, idx1a, idx2a, idx3a)
            da = (pltpu.async_copy(t04.at[idx04a], r04a, s04a),
                  pltpu.async_copy(t1.at[idx1a], r1a, s1a),
                  pltpu.async_copy(t2.at[idx2a], r2a, s2a),
                  pltpu.async_copy(t3.at[idx3a], r3a, s3a))
            idx_phase(res1, l1 * _T, idx04b, idx1b, idx2b, idx3b)
            db = (pltpu.async_copy(t04.at[idx04b], r04b, s04b),
                  pltpu.async_copy(t1.at[idx1b], r1b, s1b),
                  pltpu.async_copy(t2.at[idx2b], r2b, s2b),
                  pltpu.async_copy(t3.at[idx3b], r3b, s3b))
            for d in da:
                d.wait()
            con_phase(res0, l0, r04a, r1a, r2a, r3a)
            for d in db:
                d.wait()
            con_phase(res1, l1, r04b, r1b, r2b, r3b)
            return p * 2.25

        lax.fori_loop(0, _L // 2, pair_body, jnp.float32(16.0))
        pltpu.sync_copy(encb, enc.at[pl.ds(base * 84, _P * 84)])
        return carry

    lax.fori_loop(0, chunks, chunk_body, 0)


def _sc_encode(zin, yin, xin, tvec, t04, t1, t2, t3):
    n = zin.shape[0]
    mesh = plsc.VectorSubcoreMesh(core_axis_name="c", subcore_axis_name="s")
    dbl = []
    for _ in range(2):
        dbl += [
            pltpu.VMEM((8 * _P,), jnp.int32),
            pltpu.VMEM((4 * _P,), jnp.int32),
            pltpu.VMEM((4 * _P,), jnp.int32),
            pltpu.VMEM((4 * _P,), jnp.int32),
        ]
    rows = []
    for _ in range(2):
        rows += [
            pltpu.VMEM((8 * _P, 4), jnp.float32),
            pltpu.VMEM((4 * _P, 2), jnp.float32),
            pltpu.VMEM((4 * _P, 2), jnp.float32),
            pltpu.VMEM((4 * _P, 2), jnp.float32),
        ]
    return pl.kernel(
        _enc_body,
        out_type=jax.ShapeDtypeStruct((n * 84,), jnp.float32),
        mesh=mesh,
        compiler_params=pltpu.CompilerParams(
            needs_layout_passes=False, use_tc_tiling_on_sc=False),
        scratch_types=[
            pltpu.VMEM((_P,), jnp.float32),
            pltpu.VMEM((_P,), jnp.float32),
            pltpu.VMEM((_P,), jnp.float32),
            *dbl,
            *rows,
            pltpu.VMEM((_P * 84,), jnp.float32),
            pltpu.VMEM((16,), jnp.float32),
        ] + [pltpu.SemaphoreType.DMA] * 8,
    )(zin, yin, xin, tvec, t04, t1, t2, t3)


def _mlp_body(x_ref, w1_ref, w2_ref, w3_ref, o_ref):
    dn = (((1,), (0,)), ((), ()))
    h = jnp.maximum(lax.dot_general(
        x_ref[...], w1_ref[...], dn,
        precision=lax.Precision.HIGHEST,
        preferred_element_type=jnp.float32), 0.0)
    h = jnp.maximum(lax.dot_general(
        h, w2_ref[...], dn,
        precision=lax.Precision.HIGHEST,
        preferred_element_type=jnp.float32), 0.0)
    o_ref[...] = lax.dot_general(
        h, w3_ref[...], dn,
        precision=lax.Precision.HIGHEST,
        preferred_element_type=jnp.float32)


def _mlp(enc, W1, W2, W3):
    n = enc.shape[0]
    blk = 8192 if n % 8192 == 0 else n
    return pl.pallas_call(
        _mlp_body,
        out_shape=jax.ShapeDtypeStruct((n, 1), jnp.float32),
        grid=(n // blk,),
        in_specs=[
            pl.BlockSpec((blk, 84), lambda i: (i, 0)),
            pl.BlockSpec((84, 64), lambda i: (0, 0)),
            pl.BlockSpec((64, 64), lambda i: (0, 0)),
            pl.BlockSpec((64, 1), lambda i: (0, 0)),
        ],
        out_specs=pl.BlockSpec((blk, 1), lambda i: (i, 0)),
    )(enc, W1, W2, W3)


def kernel(zyx, t, table0, table1, table2, table3, table4, W1, W2, W3):
    zc = zyx.T
    zin, yin, xin = zc[0], zc[1], zc[2]
    t04 = jnp.concatenate([table0, table4], axis=-1).reshape(_L * _T, 4)
    t1r = table1.reshape(_L * _T, 2)
    t2r = table2.reshape(_L * _T, 2)
    t3r = table3.reshape(_L * _T, 2)
    tvec = jnp.full((16,), t, jnp.float32)
    enc = _sc_encode(zin, yin, xin, tvec, t04, t1r, t2r, t3r).reshape(-1, 84)
    return _mlp(enc, W1, W2, W3)
